# baseline (speedup 1.0000x reference)
import functools

import jax
import jax.numpy as jnp
from jax.experimental import pallas as pl


_EB = 64
_NB = 1000


def _proj_kernel(x_ref, w_ref, am_ref, h_ref, alpha_ref):
    h = jnp.dot(x_ref[...], w_ref[...], precision=jax.lax.Precision.HIGHEST,
                preferred_element_type=jnp.float32)
    h_ref[...] = h
    alpha_ref[...] = jnp.dot(h, am_ref[...],
                             precision=jax.lax.Precision.HIGHEST,
                             preferred_element_type=jnp.float32)


def _edge_kernel(h_ref, alpha_ref, r_ref, p1_ref, p2_ref, src_ref, dst_ref,
                 dstrow_ref, out_ref, den_ref, *, n_nodes):
    i = pl.program_id(0)

    @pl.when(i == 0)
    def _init():
        out_ref[...] = jnp.zeros_like(out_ref)
        den_ref[...] = jnp.zeros_like(den_ref)

    eb = src_ref.shape[0]
    hi = jax.lax.Precision.HIGHEST
    onehot_s = (jax.lax.broadcasted_iota(jnp.int32, (eb, n_nodes), 1)
                == src_ref[...]).astype(jnp.float32)
    onehot_d = (jax.lax.broadcasted_iota(jnp.int32, (eb, n_nodes), 1)
                == dst_ref[...]).astype(jnp.float32)
    onehot_dt = (jax.lax.broadcasted_iota(jnp.int32, (n_nodes, eb), 0)
                 == dstrow_ref[0]).astype(jnp.float32)

    h_src = jnp.dot(onehot_s, h_ref[...], preferred_element_type=jnp.float32)
    a_s8 = jnp.dot(onehot_s, alpha_ref[...], precision=hi,
                   preferred_element_type=jnp.float32)
    a_d8 = jnp.dot(onehot_d, alpha_ref[...], precision=hi,
                   preferred_element_type=jnp.float32)
    e = (jnp.dot(a_s8, p1_ref[...], precision=hi,
                 preferred_element_type=jnp.float32)
         + jnp.dot(a_d8, p2_ref[...], precision=hi,
                   preferred_element_type=jnp.float32))
    e = jnp.where(e >= 0.0, e, 0.2 * e)
    ex = jnp.exp(e)
    ex_rep = jnp.dot(ex, r_ref[...], precision=hi,
                     preferred_element_type=jnp.float32)
    msg = ex_rep * h_src
    out_ref[...] += jnp.dot(onehot_dt, msg, preferred_element_type=jnp.float32)
    den_ref[...] += jnp.dot(onehot_dt, ex, precision=hi,
                            preferred_element_type=jnp.float32)


def _div_kernel(num_ref, den_ref, r_ref, out_ref):
    den_rep = jnp.dot(den_ref[...], r_ref[...],
                      precision=jax.lax.Precision.HIGHEST,
                      preferred_element_type=jnp.float32)
    out_ref[...] = num_ref[...] / (den_rep + 1e-8)


def kernel(x, edge_index, W, a_src, a_dst):
    n, in_f = x.shape
    h_heads, f_out = a_src.shape
    hf = h_heads * f_out
    e_total = edge_index.shape[1]

    eye_h = jnp.eye(h_heads, dtype=jnp.float32)
    a_s_mat = (a_src[:, :, None] * eye_h[:, None, :]).reshape(hf, h_heads)
    a_d_mat = (a_dst[:, :, None] * eye_h[:, None, :]).reshape(hf, h_heads)
    am = jnp.concatenate([a_s_mat, a_d_mat], axis=1)
    r_mat = jnp.repeat(eye_h, f_out, axis=1)
    zeros_h = jnp.zeros((h_heads, h_heads), dtype=jnp.float32)
    p1 = jnp.concatenate([eye_h, zeros_h], axis=0)
    p2 = jnp.concatenate([zeros_h, eye_h], axis=0)

    h, alpha = pl.pallas_call(
        _proj_kernel,
        out_shape=(
            jax.ShapeDtypeStruct((n, hf), jnp.float32),
            jax.ShapeDtypeStruct((n, 2 * h_heads), jnp.float32),
        ),
    )(x, W, am)

    src_col = edge_index[0].astype(jnp.int32).reshape(e_total, 1)
    dst_col = edge_index[1].astype(jnp.int32).reshape(e_total, 1)
    n_steps = e_total // _EB
    dst_row = edge_index[1].astype(jnp.int32).reshape(n_steps, 1, _EB)

    num, den = pl.pallas_call(
        functools.partial(_edge_kernel, n_nodes=n),
        grid=(n_steps,),
        in_specs=[
            pl.BlockSpec((n, hf), lambda i: (0, 0)),
            pl.BlockSpec((n, 2 * h_heads), lambda i: (0, 0)),
            pl.BlockSpec((h_heads, hf), lambda i: (0, 0)),
            pl.BlockSpec((2 * h_heads, h_heads), lambda i: (0, 0)),
            pl.BlockSpec((2 * h_heads, h_heads), lambda i: (0, 0)),
            pl.BlockSpec((_EB, 1), lambda i: (i, 0)),
            pl.BlockSpec((_EB, 1), lambda i: (i, 0)),
            pl.BlockSpec((1, 1, _EB), lambda i: (i, 0, 0)),
        ],
        out_specs=(
            pl.BlockSpec((n, hf), lambda i: (0, 0)),
            pl.BlockSpec((n, h_heads), lambda i: (0, 0)),
        ),
        out_shape=(
            jax.ShapeDtypeStruct((n, hf), jnp.float32),
            jax.ShapeDtypeStruct((n, h_heads), jnp.float32),
        ),
    )(h, alpha, r_mat, p1, p2, src_col, dst_col, dst_row)

    out = pl.pallas_call(
        _div_kernel,
        grid=(n // _NB,),
        in_specs=[
            pl.BlockSpec((_NB, hf), lambda i: (i, 0)),
            pl.BlockSpec((_NB, h_heads), lambda i: (i, 0)),
            pl.BlockSpec((h_heads, hf), lambda i: (0, 0)),
        ],
        out_specs=pl.BlockSpec((_NB, hf), lambda i: (i, 0)),
        out_shape=jax.ShapeDtypeStruct((n, hf), jnp.float32),
    )(num, den, r_mat)
    return out


# speedup vs baseline: 1.4426x; 1.4426x over previous
import functools

import jax
import jax.numpy as jnp
from jax.experimental import pallas as pl


_EB = 128
_NB = 1000


def _proj_kernel(x_ref, w_ref, am_ref, h_ref, alpha_ref):
    h = jnp.dot(x_ref[...], w_ref[...], precision=jax.lax.Precision.HIGHEST,
                preferred_element_type=jnp.float32)
    h_ref[...] = h
    alpha_ref[...] = jnp.dot(h, am_ref[...],
                             precision=jax.lax.Precision.HIGHEST,
                             preferred_element_type=jnp.float32)


def _edge_kernel(h_ref, alpha_ref, r_ref, p1_ref, p2_ref, src_ref, dst_ref,
                 dstrow_ref, out_ref, den_ref, *, n_nodes):
    i = pl.program_id(0)

    @pl.when(i == 0)
    def _init():
        out_ref[...] = jnp.zeros_like(out_ref)
        den_ref[...] = jnp.zeros_like(den_ref)

    eb = src_ref.shape[0]
    hi = jax.lax.Precision.HIGHEST
    onehot_s = (jax.lax.broadcasted_iota(jnp.int32, (eb, n_nodes), 1)
                == src_ref[...]).astype(jnp.float32)
    onehot_d = (jax.lax.broadcasted_iota(jnp.int32, (eb, n_nodes), 1)
                == dst_ref[...]).astype(jnp.float32)
    onehot_dt = (jax.lax.broadcasted_iota(jnp.int32, (n_nodes, eb), 0)
                 == dstrow_ref[0]).astype(jnp.float32)

    h_src = jnp.dot(onehot_s, h_ref[...], preferred_element_type=jnp.float32)
    a_s8 = jnp.dot(onehot_s, alpha_ref[...], precision=hi,
                   preferred_element_type=jnp.float32)
    a_d8 = jnp.dot(onehot_d, alpha_ref[...], precision=hi,
                   preferred_element_type=jnp.float32)
    e = (jnp.dot(a_s8, p1_ref[...], precision=hi,
                 preferred_element_type=jnp.float32)
         + jnp.dot(a_d8, p2_ref[...], precision=hi,
                   preferred_element_type=jnp.float32))
    e = jnp.where(e >= 0.0, e, 0.2 * e)
    ex = jnp.exp(e)
    ex_rep = jnp.dot(ex, r_ref[...], precision=hi,
                     preferred_element_type=jnp.float32)
    msg = ex_rep * h_src
    out_ref[...] += jnp.dot(onehot_dt, msg, preferred_element_type=jnp.float32)
    den_ref[...] += jnp.dot(onehot_dt, ex, precision=hi,
                            preferred_element_type=jnp.float32)


def _div_kernel(num_ref, den_ref, r_ref, out_ref):
    den_rep = jnp.dot(den_ref[...], r_ref[...],
                      precision=jax.lax.Precision.HIGHEST,
                      preferred_element_type=jnp.float32)
    out_ref[...] = num_ref[...] / (den_rep + 1e-8)


def kernel(x, edge_index, W, a_src, a_dst):
    n, in_f = x.shape
    h_heads, f_out = a_src.shape
    hf = h_heads * f_out
    e_total = edge_index.shape[1]

    eye_h = jnp.eye(h_heads, dtype=jnp.float32)
    a_s_mat = (a_src[:, :, None] * eye_h[:, None, :]).reshape(hf, h_heads)
    a_d_mat = (a_dst[:, :, None] * eye_h[:, None, :]).reshape(hf, h_heads)
    am = jnp.concatenate([a_s_mat, a_d_mat], axis=1)
    r_mat = jnp.repeat(eye_h, f_out, axis=1)
    zeros_h = jnp.zeros((h_heads, h_heads), dtype=jnp.float32)
    p1 = jnp.concatenate([eye_h, zeros_h], axis=0)
    p2 = jnp.concatenate([zeros_h, eye_h], axis=0)

    h, alpha = pl.pallas_call(
        _proj_kernel,
        out_shape=(
            jax.ShapeDtypeStruct((n, hf), jnp.float32),
            jax.ShapeDtypeStruct((n, 2 * h_heads), jnp.float32),
        ),
    )(x, W, am)

    src_col = edge_index[0].astype(jnp.int32).reshape(e_total, 1)
    dst_col = edge_index[1].astype(jnp.int32).reshape(e_total, 1)
    n_steps = e_total // _EB
    dst_row = edge_index[1].astype(jnp.int32).reshape(n_steps, 1, _EB)

    num, den = pl.pallas_call(
        functools.partial(_edge_kernel, n_nodes=n),
        grid=(n_steps,),
        in_specs=[
            pl.BlockSpec((n, hf), lambda i: (0, 0)),
            pl.BlockSpec((n, 2 * h_heads), lambda i: (0, 0)),
            pl.BlockSpec((h_heads, hf), lambda i: (0, 0)),
            pl.BlockSpec((2 * h_heads, h_heads), lambda i: (0, 0)),
            pl.BlockSpec((2 * h_heads, h_heads), lambda i: (0, 0)),
            pl.BlockSpec((_EB, 1), lambda i: (i, 0)),
            pl.BlockSpec((_EB, 1), lambda i: (i, 0)),
            pl.BlockSpec((1, 1, _EB), lambda i: (i, 0, 0)),
        ],
        out_specs=(
            pl.BlockSpec((n, hf), lambda i: (0, 0)),
            pl.BlockSpec((n, h_heads), lambda i: (0, 0)),
        ),
        out_shape=(
            jax.ShapeDtypeStruct((n, hf), jnp.float32),
            jax.ShapeDtypeStruct((n, h_heads), jnp.float32),
        ),
    )(h, alpha, r_mat, p1, p2, src_col, dst_col, dst_row)

    out = pl.pallas_call(
        _div_kernel,
        grid=(n // _NB,),
        in_specs=[
            pl.BlockSpec((_NB, hf), lambda i: (i, 0)),
            pl.BlockSpec((_NB, h_heads), lambda i: (i, 0)),
            pl.BlockSpec((h_heads, hf), lambda i: (0, 0)),
        ],
        out_specs=pl.BlockSpec((_NB, hf), lambda i: (i, 0)),
        out_shape=jax.ShapeDtypeStruct((n, hf), jnp.float32),
    )(num, den, r_mat)
    return out
